# baseline (speedup 1.0000x reference)
import functools

import jax
import jax.numpy as jnp
from jax import lax
from jax.experimental import pallas as pl
from jax.experimental.pallas import tpu as pltpu
from jax.experimental.pallas import tpu_sc as plsc

_STORAGE_SIZE = 100000
_CAMS = 8
_FEATS = 7
_BATCH = 16384

_NW = 32
_FRAMES_W = _BATCH // _NW
_LOOK_W = _FRAMES_W * _CAMS
_CHUNK = 64
_NCHUNK = _LOOK_W // _CHUNK
_L = 16
_NROW8 = _STORAGE_SIZE * _CAMS * _FEATS // 8
_QMAX = _NROW8 - 1


def _sc_body(frame_hbm, table8_hbm, rot_hbm, trans_hbm, focal_hbm,
             fidx_v, idx2d, s_st, rows_v, rot_st, trans_st, focal_st, sem):
    wid = lax.axis_index("s") * 2 + lax.axis_index("c")

    iota = lax.iota(jnp.int32, _L)
    sel = lax.shift_right_logical(iota, 3)
    camoff = (iota & 7) * _STORAGE_SIZE

    pltpu.sync_copy(frame_hbm.at[pl.ds(wid * _FRAMES_W, _FRAMES_W)], fidx_v)

    def build(c, carry):
        for u in range(4):
            pc = u * _L
            p0 = c * _CHUNK + pc
            frames = lax.shift_right_logical(p0, 3) + sel
            fvals = plsc.load_gather(fidx_v, [frames])
            t = (fvals + camoff) * _FEATS
            q = lax.shift_right_logical(t, 3)
            s_st[pl.ds(p0, _L)] = t & 7
            rowv = iota * 0 + c
            colv = pc * 2 + iota * 2
            plsc.store_scatter(idx2d, [rowv, colv], q)
            plsc.store_scatter(idx2d, [rowv, colv + 1],
                               jnp.minimum(q + 1, _QMAX))
        return carry

    lax.fori_loop(0, _NCHUNK, build, 0)

    def chunk(c, carry):
        pltpu.async_copy(table8_hbm.at[idx2d.at[c]], rows_v, sem).wait()
        for k in range(12):
            p = k * _L + iota
            l = lax.shift_right_logical(p * 171, 9)
            comp = p - l * 3
            sv = plsc.load_gather(s_st, [c * _CHUNK + l])
            u = sv + comp
            rot_st[pl.ds(c * 192 + k * _L, _L)] = plsc.load_gather(
                rows_v, [2 * l + lax.shift_right_logical(u, 3), u & 7])
            u2 = u + 3
            trans_st[pl.ds(c * 192 + k * _L, _L)] = plsc.load_gather(
                rows_v, [2 * l + lax.shift_right_logical(u2, 3), u2 & 7]
            ) * 10.0
        for k in range(4):
            p = k * _L + iota
            sv = plsc.load_gather(s_st, [c * _CHUNK + p])
            u = sv + 6
            focal_st[pl.ds(c * _CHUNK + k * _L, _L)] = plsc.load_gather(
                rows_v, [2 * p + lax.shift_right_logical(u, 3), u & 7]
            ) * 1000.0
        return carry

    lax.fori_loop(0, _NCHUNK, chunk, 0)

    pltpu.sync_copy(rot_st, rot_hbm.at[pl.ds(wid * _LOOK_W * 3, _LOOK_W * 3)])
    pltpu.sync_copy(trans_st, trans_hbm.at[pl.ds(wid * _LOOK_W * 3, _LOOK_W * 3)])
    pltpu.sync_copy(focal_st, focal_hbm.at[pl.ds(wid * _LOOK_W, _LOOK_W)])


@jax.jit
def _sc_call(frame_indexes, storage):
    mesh = plsc.VectorSubcoreMesh(core_axis_name="c", subcore_axis_name="s")
    f = functools.partial(
        pl.kernel,
        mesh=mesh,
        out_type=[
            jax.ShapeDtypeStruct((_BATCH * _CAMS * 3,), jnp.float32),
            jax.ShapeDtypeStruct((_BATCH * _CAMS * 3,), jnp.float32),
            jax.ShapeDtypeStruct((_BATCH * _CAMS,), jnp.float32),
        ],
        scratch_types=[
            pltpu.VMEM((_FRAMES_W,), jnp.int32),
            pltpu.VMEM((_NCHUNK, 2 * _CHUNK), jnp.int32),
            pltpu.VMEM((_LOOK_W,), jnp.int32),
            pltpu.VMEM((2 * _CHUNK, 8), jnp.float32),
            pltpu.VMEM((_LOOK_W * 3,), jnp.float32),
            pltpu.VMEM((_LOOK_W * 3,), jnp.float32),
            pltpu.VMEM((_LOOK_W,), jnp.float32),
            pltpu.SemaphoreType.DMA,
        ],
        compiler_params=pltpu.CompilerParams(
            use_tc_tiling_on_sc=False, needs_layout_passes=False),
    )(_sc_body)
    table8 = storage.reshape(_NROW8, 8)
    return f(frame_indexes, table8)


def kernel(frame_indexes, storage):
    rot, trans, focal = _sc_call(frame_indexes, storage)
    return (rot.reshape(_BATCH, _CAMS, 3),
            trans.reshape(_BATCH, _CAMS, 3),
            focal.reshape(_BATCH, _CAMS))


# speedup vs baseline: 1.0749x; 1.0749x over previous
import functools

import jax
import jax.numpy as jnp
from jax import lax
from jax.experimental import pallas as pl
from jax.experimental.pallas import tpu as pltpu
from jax.experimental.pallas import tpu_sc as plsc

_STORAGE_SIZE = 100000
_CAMS = 8
_FEATS = 7
_BATCH = 16384

_NW = 32
_FRAMES_W = _BATCH // _NW
_LOOK_W = _FRAMES_W * _CAMS
_CHUNK = 64
_NCHUNK = _LOOK_W // _CHUNK
_L = 16
_NROW8 = _STORAGE_SIZE * _CAMS * _FEATS // 8
_QMAX = _NROW8 - 1


def _sc_body(frame_hbm, table8_hbm, rot_hbm, trans_hbm, focal_hbm,
             fidx_v, idx2d, s_st, rows_v, p0_st, p1_st, p2_st,
             rot_st, trans_st, focal_st, sem):
    wid = lax.axis_index("s") * 2 + lax.axis_index("c")

    iota = lax.iota(jnp.int32, _L)
    sel = lax.shift_right_logical(iota, 3)
    camoff = (iota & 7) * _STORAGE_SIZE

    pltpu.sync_copy(frame_hbm.at[pl.ds(wid * _FRAMES_W, _FRAMES_W)], fidx_v)

    for k in range(12):
        q = k * _L + iota
        q3 = lax.shift_right_logical(q * 171, 9)
        q24 = lax.shift_right_logical(q3, 3)
        p0_st[pl.ds(k * _L, _L)] = q24
        p1_st[pl.ds(k * _L, _L)] = q3 - q24 * 8
        p2_st[pl.ds(k * _L, _L)] = q - q3 * 3

    def build(c, carry):
        for u in range(4):
            pc = u * _L
            p0 = c * _CHUNK + pc
            frames = lax.shift_right_logical(p0, 3) + sel
            fvals = plsc.load_gather(fidx_v, [frames])
            t = (fvals + camoff) * _FEATS
            q = lax.shift_right_logical(t, 3)
            s_st[pl.ds(p0, _L)] = t & 7
            rowv = iota * 0 + c
            colv = pc * 2 + iota * 2
            plsc.store_scatter(idx2d, [rowv, colv], q)
            plsc.store_scatter(idx2d, [rowv, colv + 1],
                               jnp.minimum(q + 1, _QMAX))
        return carry

    lax.fori_loop(0, _NCHUNK, build, 0)

    def chunk(c, carry):
        pltpu.async_copy(table8_hbm.at[idx2d.at[c]], rows_v, sem).wait()
        c8 = c * 8
        for k in range(12):
            p = k * _L + iota
            l = lax.shift_right_logical(p * 171, 9)
            comp = p - l * 3
            sv = plsc.load_gather(s_st, [c * _CHUNK + l])
            u = sv + comp
            i0 = p0_st[pl.ds(k * _L, _L)] + c8
            i1 = p1_st[pl.ds(k * _L, _L)]
            i2 = p2_st[pl.ds(k * _L, _L)]
            rotv = plsc.load_gather(
                rows_v, [2 * l + lax.shift_right_logical(u, 3), u & 7])
            plsc.store_scatter(rot_st, [i0, i1, i2], rotv)
            u2 = u + 3
            transv = plsc.load_gather(
                rows_v, [2 * l + lax.shift_right_logical(u2, 3), u2 & 7]
            ) * 10.0
            plsc.store_scatter(trans_st, [i0, i1, i2], transv)
        for k in range(4):
            p = k * _L + iota
            sv = plsc.load_gather(s_st, [c * _CHUNK + p])
            u = sv + 6
            focv = plsc.load_gather(
                rows_v, [2 * p + lax.shift_right_logical(u, 3), u & 7]
            ) * 1000.0
            g = c * _CHUNK + p
            plsc.store_scatter(focal_st,
                               [lax.shift_right_logical(g, 3), g & 7], focv)
        return carry

    lax.fori_loop(0, _NCHUNK, chunk, 0)

    f0 = wid * _FRAMES_W
    pltpu.sync_copy(rot_st, rot_hbm.at[pl.ds(f0, _FRAMES_W)])
    pltpu.sync_copy(trans_st, trans_hbm.at[pl.ds(f0, _FRAMES_W)])
    pltpu.sync_copy(focal_st, focal_hbm.at[pl.ds(f0, _FRAMES_W)])


@jax.jit
def _sc_call(frame_indexes, storage):
    mesh = plsc.VectorSubcoreMesh(core_axis_name="c", subcore_axis_name="s")
    f = functools.partial(
        pl.kernel,
        mesh=mesh,
        out_type=[
            jax.ShapeDtypeStruct((_BATCH, _CAMS, 3), jnp.float32),
            jax.ShapeDtypeStruct((_BATCH, _CAMS, 3), jnp.float32),
            jax.ShapeDtypeStruct((_BATCH, _CAMS), jnp.float32),
        ],
        scratch_types=[
            pltpu.VMEM((_FRAMES_W,), jnp.int32),
            pltpu.VMEM((_NCHUNK, 2 * _CHUNK), jnp.int32),
            pltpu.VMEM((_LOOK_W,), jnp.int32),
            pltpu.VMEM((2 * _CHUNK, 8), jnp.float32),
            pltpu.VMEM((192,), jnp.int32),
            pltpu.VMEM((192,), jnp.int32),
            pltpu.VMEM((192,), jnp.int32),
            pltpu.VMEM((_FRAMES_W, _CAMS, 3), jnp.float32),
            pltpu.VMEM((_FRAMES_W, _CAMS, 3), jnp.float32),
            pltpu.VMEM((_FRAMES_W, _CAMS), jnp.float32),
            pltpu.SemaphoreType.DMA,
        ],
        compiler_params=pltpu.CompilerParams(
            use_tc_tiling_on_sc=False, needs_layout_passes=False),
    )(_sc_body)
    return f(frame_indexes, storage.reshape(_NROW8, 8))


def kernel(frame_indexes, storage):
    return _sc_call(frame_indexes, storage)
